# Initial kernel scaffold; baseline (speedup 1.0000x reference)
import jax
import jax.numpy as jnp
from jax.experimental import pallas as pl


def kernel(features, codebook, W_ih, W_hh, b_ih, b_hh, W_proj, b_proj):
    raise NotImplementedError("write your pallas kernel here")


# speedup vs baseline: 2.3756x; 2.3756x over previous
import jax
import jax.numpy as jnp
from jax.experimental import pallas as pl
from jax.experimental.pallas import tpu as pltpu
from jax.experimental.pallas import tpu_sc as plsc

B, L, D = 16, 64, 256
K = 8192
N = B * L
KT = 1024
NKT = K // KT
LAMBDA_VQ = 0.25
GW = 32


def _vq_body(x2_ref, flat_ref, cb_ref, tok_ref, min_ref, arg_ref):
    j = pl.program_id(0)

    @pl.when(j == 0)
    def _init():
        min_ref[...] = jnp.full((N, 1), jnp.inf, jnp.float32)
        arg_ref[...] = jnp.zeros((N, 1), jnp.int32)

    scores = jax.lax.dot_general(
        flat_ref[...], cb_ref[...], (((1,), (1,)), ((), ())),
        preferred_element_type=jnp.float32)
    cb = cb_ref[...]
    c2 = jnp.sum(cb * cb, axis=1)[None, :]
    d2 = jnp.sqrt(jnp.maximum((x2_ref[...] - 2.0 * scores) + c2, 0.0))
    tile_min = jnp.min(d2, axis=1, keepdims=True)
    iota = jax.lax.broadcasted_iota(jnp.int32, (N, KT), 1)
    masked = jnp.where(d2 == tile_min, iota, KT)
    tile_arg = jnp.min(masked, axis=1, keepdims=True) + j * KT
    better = tile_min < min_ref[...]
    arg_ref[...] = jnp.where(better, tile_arg, arg_ref[...])
    min_ref[...] = jnp.where(better, tile_min, min_ref[...])

    @pl.when(j == NKT - 1)
    def _write():
        tok_ref[...] = arg_ref[...]


def _vq_tokens(x2, flat, codebook):
    return pl.pallas_call(
        _vq_body,
        grid=(NKT,),
        in_specs=[
            pl.BlockSpec((N, 1), lambda j: (0, 0)),
            pl.BlockSpec((N, D), lambda j: (0, 0)),
            pl.BlockSpec((KT, D), lambda j: (j, 0)),
        ],
        out_specs=pl.BlockSpec((N, 1), lambda j: (0, 0)),
        out_shape=jax.ShapeDtypeStruct((N, 1), jnp.int32),
        scratch_shapes=[
            pltpu.VMEM((N, 1), jnp.float32),
            pltpu.VMEM((N, 1), jnp.int32),
        ],
    )(x2, flat, codebook)


def _sc_gather(codebook, tokens_row):
    mesh = plsc.VectorSubcoreMesh(core_axis_name="core",
                                  subcore_axis_name="subcore")

    @pl.kernel(out_type=jax.ShapeDtypeStruct((N, D), codebook.dtype),
               mesh=mesh)
    def gather_kernel(cb_hbm, i_hbm, o_hbm):
        def body(i_vmem, o_vmem):
            pltpu.sync_copy(cb_hbm.at[i_vmem.at[0]], o_vmem)

        pltpu.emit_pipeline(
            body,
            grid=(N // GW,),
            in_specs=[pl.BlockSpec((1, GW), index_map=lambda i: (i, 0))],
            out_specs=[pl.BlockSpec((GW, D), index_map=lambda i: (i, 0))],
            core_axis_name=("core", "subcore"),
            dimension_semantics=(pltpu.PARALLEL,),
        )(i_hbm, o_hbm)

    return gather_kernel(codebook, tokens_row)


def _gru_body(q_ref, f_ref, fr_ref, wih_ref, whh_ref, bih_ref, bhh_ref,
              wp_ref, bp_ref, cp_ref, com_ref, tot_ref, gi_ref, ys_ref):
    q2 = q_ref[...]
    gi_ref[...] = jax.lax.dot_general(
        q2, wih_ref[...], (((1,), (1,)), ((), ())),
        preferred_element_type=jnp.float32) + bih_ref[...]
    whh = whh_ref[...]
    bhh = bhh_ref[...]

    def step(t, h):
        gi = gi_ref[pl.ds(t * B, B), :]
        gh = jax.lax.dot_general(
            h, whh, (((1,), (1,)), ((), ())),
            preferred_element_type=jnp.float32) + bhh
        r = jax.nn.sigmoid(gi[:, :D] + gh[:, :D])
        z = jax.nn.sigmoid(gi[:, D:2 * D] + gh[:, D:2 * D])
        n = jnp.tanh(gi[:, 2 * D:] + r * gh[:, 2 * D:])
        h_new = (1.0 - z) * n + z * h
        ys_ref[pl.ds(t * B, B), :] = h_new
        return h_new

    jax.lax.fori_loop(0, L, step, jnp.zeros((B, D), jnp.float32))

    ctx = jax.lax.dot_general(
        ys_ref[...], wp_ref[...], (((1,), (1,)), ((), ())),
        preferred_element_type=jnp.float32) + bp_ref[...]
    f2 = f_ref[...]
    fr2 = fr_ref[...]
    acc = 0.0
    for k in (1, 2, 3):
        m = (L - k) * B
        pos = jnp.sum(ctx[:m] * f2[k * B:], axis=1, keepdims=True)
        neg = jnp.sum(ctx[:m] * fr2[k * B:], axis=1, keepdims=True)
        lk = -jnp.log(jax.nn.sigmoid(pos)) \
             - LAMBDA_VQ * jnp.log(1.0 - jax.nn.sigmoid(neg))
        acc = acc + jnp.mean(lk)
    cp = acc / 3.0
    diff = f2 - q2
    com = jnp.mean(diff * diff)
    cp_ref[...] = cp.reshape(1, 1)
    com_ref[...] = com.reshape(1, 1)
    tot_ref[...] = ((cp + com) + LAMBDA_VQ * com).reshape(1, 1)


def _gru_loss(q2, f2, fr2, W_ih, W_hh, bih, bhh, W_proj, bp):
    out_shape = [jax.ShapeDtypeStruct((1, 1), jnp.float32)] * 3
    return pl.pallas_call(
        _gru_body,
        in_specs=[pl.BlockSpec(a.shape, lambda: (0,) * a.ndim)
                  for a in (q2, f2, fr2, W_ih, W_hh, bih, bhh, W_proj, bp)],
        out_specs=[pl.BlockSpec((1, 1), lambda: (0, 0))] * 3,
        out_shape=out_shape,
        scratch_shapes=[
            pltpu.VMEM((N, 3 * D), jnp.float32),
            pltpu.VMEM((N, D), jnp.float32),
        ],
    )(q2, f2, fr2, W_ih, W_hh, bih, bhh, W_proj, bp)


def kernel(features, codebook, W_ih, W_hh, b_ih, b_hh, W_proj, b_proj):
    flat = features.reshape(-1, D)
    x2 = jnp.sum(flat * flat, axis=-1, keepdims=True)
    tokens = _vq_tokens(x2, flat, codebook)
    token_indices = tokens.reshape(B, L)
    quantized = _sc_gather(codebook, tokens.reshape(N // GW, GW))
    q3 = quantized.reshape(B, L, D)

    qT = jnp.swapaxes(q3, 0, 1).reshape(N, D)
    fT = jnp.swapaxes(features, 0, 1)
    frT = jnp.roll(fT, 1, axis=1).reshape(N, D)
    cp, com, tot = _gru_loss(
        qT, fT.reshape(N, D), frT, W_ih, W_hh,
        b_ih.reshape(1, -1), b_hh.reshape(1, -1), W_proj, b_proj.reshape(1, -1))
    cp = cp.reshape(())
    com = com.reshape(())
    tot = tot.reshape(())
    return (token_indices, q3, cp, com, com, tot)


# speedup vs baseline: 2.6109x; 1.0991x over previous
import jax
import jax.numpy as jnp
from jax.experimental import pallas as pl
from jax.experimental.pallas import tpu as pltpu
from jax.experimental.pallas import tpu_sc as plsc

B, L, D = 16, 64, 256
K = 8192
N = B * L
KT = 1024
NKT = K // KT
LAMBDA_VQ = 0.25
GW = 32


def _vq_body(x2_ref, flat_ref, cb_ref, tok_ref, min_ref, arg_ref):
    j = pl.program_id(0)

    @pl.when(j == 0)
    def _init():
        min_ref[...] = jnp.full((N, 1), jnp.inf, jnp.float32)
        arg_ref[...] = jnp.zeros((N, 1), jnp.int32)

    scores = jax.lax.dot_general(
        flat_ref[...], cb_ref[...], (((1,), (1,)), ((), ())),
        preferred_element_type=jnp.float32)
    cb = cb_ref[...]
    c2 = jnp.sum(cb * cb, axis=1)[None, :]
    d2 = jnp.maximum((x2_ref[...] - 2.0 * scores) + c2, 0.0)
    dist = d2 * jax.lax.rsqrt(d2)
    tile_min = jnp.min(dist, axis=1, keepdims=True)
    iota = jax.lax.broadcasted_iota(jnp.int32, (N, KT), 1)
    masked = jnp.where(dist == tile_min, iota, KT)
    tile_arg = jnp.min(masked, axis=1, keepdims=True) + j * KT
    better = tile_min < min_ref[...]
    arg_ref[...] = jnp.where(better, tile_arg, arg_ref[...])
    min_ref[...] = jnp.where(better, tile_min, min_ref[...])

    @pl.when(j == NKT - 1)
    def _write():
        tok_ref[...] = arg_ref[...]


def _vq_tokens(x2, flat, codebook):
    return pl.pallas_call(
        _vq_body,
        grid=(NKT,),
        in_specs=[
            pl.BlockSpec((N, 1), lambda j: (0, 0)),
            pl.BlockSpec((N, D), lambda j: (0, 0)),
            pl.BlockSpec((KT, D), lambda j: (j, 0)),
        ],
        out_specs=pl.BlockSpec((N, 1), lambda j: (0, 0)),
        out_shape=jax.ShapeDtypeStruct((N, 1), jnp.int32),
        scratch_shapes=[
            pltpu.VMEM((N, 1), jnp.float32),
            pltpu.VMEM((N, 1), jnp.int32),
        ],
    )(x2, flat, codebook)


def _sc_gather(codebook, tokens_rows):
    mesh = plsc.VectorSubcoreMesh(core_axis_name="core",
                                  subcore_axis_name="subcore")

    @pl.kernel(out_type=jax.ShapeDtypeStruct((N, D), codebook.dtype),
               mesh=mesh)
    def gather_kernel(cb_hbm, i_hbm, o_hbm):
        def body(i_vmem, o_vmem):
            pltpu.sync_copy(cb_hbm.at[i_vmem.at[0]], o_vmem)

        pltpu.emit_pipeline(
            body,
            grid=(N // GW,),
            in_specs=[pl.BlockSpec((1, GW), index_map=lambda i: (i, 0))],
            out_specs=[pl.BlockSpec((GW, D), index_map=lambda i: (i, 0))],
            core_axis_name=("core", "subcore"),
            dimension_semantics=(pltpu.PARALLEL,),
        )(i_hbm, o_hbm)

    return gather_kernel(codebook, tokens_rows)


def _gru_body(q_ref, f_ref, wih_ref, whh_ref, bih_ref, bhh_ref,
              wp_ref, bp_ref, cp_ref, com_ref, tot_ref, gi_ref, ys_ref):
    q2 = q_ref[...]
    gi = jax.lax.dot_general(
        q2.astype(jnp.bfloat16), wih_ref[...].astype(jnp.bfloat16),
        (((1,), (1,)), ((), ())),
        preferred_element_type=jnp.float32) + bih_ref[...]
    gi_ref[...] = gi.reshape(L, B, 3 * D)
    whh_b = whh_ref[...].astype(jnp.bfloat16)
    bhh = bhh_ref[...]

    def step(t, h):
        gi_t = gi_ref[t]
        gh = jax.lax.dot_general(
            h.astype(jnp.bfloat16), whh_b, (((1,), (1,)), ((), ())),
            preferred_element_type=jnp.float32) + bhh
        r = jax.nn.sigmoid(gi_t[:, :D] + gh[:, :D])
        z = jax.nn.sigmoid(gi_t[:, D:2 * D] + gh[:, D:2 * D])
        n = jnp.tanh(gi_t[:, 2 * D:] + r * gh[:, 2 * D:])
        h_new = (1.0 - z) * n + z * h
        ys_ref[t] = h_new
        return h_new

    jax.lax.fori_loop(0, L, step, jnp.zeros((B, D), jnp.float32))

    ctx = jax.lax.dot_general(
        ys_ref[...].reshape(N, D).astype(jnp.bfloat16),
        wp_ref[...].astype(jnp.bfloat16), (((1,), (1,)), ((), ())),
        preferred_element_type=jnp.float32) + bp_ref[...]
    f3 = f_ref[...]
    f2 = f3.reshape(N, D)
    fr2 = jnp.concatenate([f3[:, B - 1:], f3[:, :B - 1]], axis=1).reshape(N, D)
    acc = 0.0
    for k in (1, 2, 3):
        m = (L - k) * B
        pos = jnp.sum(ctx[:m] * f2[k * B:], axis=1, keepdims=True)
        neg = jnp.sum(ctx[:m] * fr2[k * B:], axis=1, keepdims=True)
        lk = -jnp.log(jax.nn.sigmoid(pos)) \
             - LAMBDA_VQ * jnp.log(1.0 - jax.nn.sigmoid(neg))
        acc = acc + jnp.mean(lk)
    cp = acc / 3.0
    diff = f2 - q2
    com = jnp.mean(diff * diff)
    cp_ref[...] = cp.reshape(1, 1)
    com_ref[...] = com.reshape(1, 1)
    tot_ref[...] = ((cp + com) + LAMBDA_VQ * com).reshape(1, 1)


def _gru_loss(qT, fT, W_ih, W_hh, bih, bhh, W_proj, bp):
    args = (qT, fT, W_ih, W_hh, bih, bhh, W_proj, bp)
    return pl.pallas_call(
        _gru_body,
        in_specs=[pl.BlockSpec(a.shape, lambda *_, _nd=a.ndim: (0,) * _nd)
                  for a in args],
        out_specs=[pl.BlockSpec((1, 1), lambda: (0, 0))] * 3,
        out_shape=[jax.ShapeDtypeStruct((1, 1), jnp.float32)] * 3,
        scratch_shapes=[
            pltpu.VMEM((L, B, 3 * D), jnp.float32),
            pltpu.VMEM((L, B, D), jnp.float32),
        ],
    )(*args)


def kernel(features, codebook, W_ih, W_hh, b_ih, b_hh, W_proj, b_proj):
    flat = features.reshape(-1, D)
    x2 = jnp.sum(flat * flat, axis=-1, keepdims=True)
    tokens = _vq_tokens(x2, flat, codebook)
    token_indices = tokens.reshape(B, L)
    tokens_tm = token_indices.T.reshape(N // GW, GW)
    qT = _sc_gather(codebook, tokens_tm)
    q3 = qT.reshape(L, B, D).transpose(1, 0, 2)
    fT = features.transpose(1, 0, 2)
    cp, com, tot = _gru_loss(
        qT, fT, W_ih, W_hh,
        b_ih.reshape(1, -1), b_hh.reshape(1, -1), W_proj, b_proj.reshape(1, -1))
    cp = cp.reshape(())
    com = com.reshape(())
    tot = tot.reshape(())
    return (token_indices, q3, cp, com, com, tot)


# speedup vs baseline: 2.8536x; 1.0930x over previous
import jax
import jax.numpy as jnp
from jax.experimental import pallas as pl
from jax.experimental.pallas import tpu as pltpu
from jax.experimental.pallas import tpu_sc as plsc

B, L, D = 16, 64, 256
K = 8192
N = B * L
KT = 1024
NKT = K // KT
LAMBDA_VQ = 0.25
GW = 32


def _vq_body(x2_ref, flat_ref, cb_ref, tok_ref, min_ref, arg_ref):
    j = pl.program_id(0)

    @pl.when(j == 0)
    def _init():
        min_ref[...] = jnp.full((N, 1), jnp.inf, jnp.float32)
        arg_ref[...] = jnp.zeros((N, 1), jnp.int32)

    scores = jax.lax.dot_general(
        flat_ref[...], cb_ref[...], (((1,), (1,)), ((), ())),
        preferred_element_type=jnp.float32)
    cb = cb_ref[...]
    c2 = jnp.sum(cb * cb, axis=1)[None, :]
    d2 = jnp.maximum((x2_ref[...] - 2.0 * scores) + c2, 0.0)
    dist = d2 * jax.lax.rsqrt(d2)
    tile_min = jnp.min(dist, axis=1, keepdims=True)
    iota = jax.lax.broadcasted_iota(jnp.int32, (N, KT), 1)
    masked = jnp.where(dist == tile_min, iota, KT)
    tile_arg = jnp.min(masked, axis=1, keepdims=True) + j * KT
    better = tile_min < min_ref[...]
    arg_ref[...] = jnp.where(better, tile_arg, arg_ref[...])
    min_ref[...] = jnp.where(better, tile_min, min_ref[...])

    @pl.when(j == NKT - 1)
    def _write():
        tok_ref[...] = arg_ref[...]


def _vq_tokens(x2, flat, codebook):
    return pl.pallas_call(
        _vq_body,
        grid=(NKT,),
        in_specs=[
            pl.BlockSpec((N, 1), lambda j: (0, 0)),
            pl.BlockSpec((N, D), lambda j: (0, 0)),
            pl.BlockSpec((KT, D), lambda j: (j, 0)),
        ],
        out_specs=pl.BlockSpec((N, 1), lambda j: (0, 0)),
        out_shape=jax.ShapeDtypeStruct((N, 1), jnp.int32),
        scratch_shapes=[
            pltpu.VMEM((N, 1), jnp.float32),
            pltpu.VMEM((N, 1), jnp.int32),
        ],
    )(x2, flat, codebook)


def _sc_gather(codebook, tokens_rows):
    mesh = plsc.VectorSubcoreMesh(core_axis_name="core",
                                  subcore_axis_name="subcore")

    @pl.kernel(out_type=jax.ShapeDtypeStruct((N, D), codebook.dtype),
               mesh=mesh)
    def gather_kernel(cb_hbm, i_hbm, o_hbm):
        def body(i_vmem, o_vmem):
            pltpu.sync_copy(cb_hbm.at[i_vmem.at[0]], o_vmem)

        pltpu.emit_pipeline(
            body,
            grid=(N // GW,),
            in_specs=[pl.BlockSpec((1, GW), index_map=lambda i: (i, 0))],
            out_specs=[pl.BlockSpec((GW, D), index_map=lambda i: (i, 0))],
            core_axis_name=("core", "subcore"),
            dimension_semantics=(pltpu.PARALLEL,),
        )(i_hbm, o_hbm)

    return gather_kernel(codebook, tokens_rows)


def _gru_body(q_ref, f_ref, wih_ref, whh_ref, bih_ref, bhh_ref,
              wp_ref, bp_ref, cp_ref, com_ref, tot_ref, q3_ref, gi_ref, ys_ref):
    q2 = q_ref[...]
    q3_ref[...] = q2.reshape(L, B, D).transpose(1, 0, 2)
    gi = jax.lax.dot_general(
        q2.astype(jnp.bfloat16), wih_ref[...].astype(jnp.bfloat16),
        (((1,), (1,)), ((), ())),
        preferred_element_type=jnp.float32) + bih_ref[...]
    gi_ref[...] = gi.reshape(L, B, 3 * D)
    whh_b = whh_ref[...].astype(jnp.bfloat16)
    bhh = bhh_ref[...]

    def cell(t, h):
        gi_t = gi_ref[t]
        gh = jax.lax.dot_general(
            h.astype(jnp.bfloat16), whh_b, (((1,), (1,)), ((), ())),
            preferred_element_type=jnp.float32) + bhh
        r = jax.nn.sigmoid(gi_t[:, :D] + gh[:, :D])
        z = jax.nn.sigmoid(gi_t[:, D:2 * D] + gh[:, D:2 * D])
        n = jnp.tanh(gi_t[:, 2 * D:] + r * gh[:, 2 * D:])
        h_new = (1.0 - z) * n + z * h
        ys_ref[t] = h_new
        return h_new

    def step4(i, h):
        h = cell(4 * i + 1, cell(4 * i, h))
        return cell(4 * i + 3, cell(4 * i + 2, h))

    jax.lax.fori_loop(0, L // 4, step4, jnp.zeros((B, D), jnp.float32))

    ctx = jax.lax.dot_general(
        ys_ref[...].reshape(N, D).astype(jnp.bfloat16),
        wp_ref[...].astype(jnp.bfloat16), (((1,), (1,)), ((), ())),
        preferred_element_type=jnp.float32) + bp_ref[...]
    f3 = f_ref[...].transpose(1, 0, 2)
    f2 = f3.reshape(N, D)
    fr2 = jnp.concatenate([f3[:, B - 1:], f3[:, :B - 1]], axis=1).reshape(N, D)
    acc = 0.0
    for k in (1, 2, 3):
        m = (L - k) * B
        pos = jnp.sum(ctx[:m] * f2[k * B:], axis=1, keepdims=True)
        neg = jnp.sum(ctx[:m] * fr2[k * B:], axis=1, keepdims=True)
        lk = -jnp.log(jax.nn.sigmoid(pos)) \
             - LAMBDA_VQ * jnp.log(1.0 - jax.nn.sigmoid(neg))
        acc = acc + jnp.mean(lk)
    cp = acc / 3.0
    diff = f2 - q2
    com = jnp.mean(diff * diff)
    cp_ref[...] = cp.reshape(1, 1)
    com_ref[...] = com.reshape(1, 1)
    tot_ref[...] = ((cp + com) + LAMBDA_VQ * com).reshape(1, 1)


def _gru_loss(qT, fT, W_ih, W_hh, bih, bhh, W_proj, bp):
    args = (qT, fT, W_ih, W_hh, bih, bhh, W_proj, bp)
    return pl.pallas_call(
        _gru_body,
        in_specs=[pl.BlockSpec(a.shape, lambda *_, _nd=a.ndim: (0,) * _nd)
                  for a in args],
        out_specs=[pl.BlockSpec((1, 1), lambda: (0, 0))] * 3
                  + [pl.BlockSpec((B, L, D), lambda: (0, 0, 0))],
        out_shape=[jax.ShapeDtypeStruct((1, 1), jnp.float32)] * 3
                  + [jax.ShapeDtypeStruct((B, L, D), jnp.float32)],
        scratch_shapes=[
            pltpu.VMEM((L, B, 3 * D), jnp.float32),
            pltpu.VMEM((L, B, D), jnp.float32),
        ],
    )(*args)


def kernel(features, codebook, W_ih, W_hh, b_ih, b_hh, W_proj, b_proj):
    flat = features.reshape(-1, D)
    x2 = jnp.sum(flat * flat, axis=-1, keepdims=True)
    tokens = _vq_tokens(x2, flat, codebook)
    token_indices = tokens.reshape(B, L)
    tokens_tm = token_indices.T.reshape(N // GW, GW)
    qT = _sc_gather(codebook, tokens_tm)
    cp, com, tot, q3 = _gru_loss(
        qT, features, W_ih, W_hh,
        b_ih.reshape(1, -1), b_hh.reshape(1, -1), W_proj, b_proj.reshape(1, -1))
    cp = cp.reshape(())
    com = com.reshape(())
    tot = tot.reshape(())
    return (token_indices, q3, cp, com, com, tot)
